# baseline (speedup 1.0000x reference)
import functools

import jax
import jax.numpy as jnp
from jax import lax
from jax.experimental import pallas as pl
from jax.experimental.pallas import tpu as pltpu
from jax.experimental.pallas import tpu_sc as plsc

NUM_TOKENS = 16384
MODEL_DIM = 2048
NUM_EXPERTS = 64
LANES = 16
NUM_CORES = 2
NUM_SUBCORES = 16
NUM_WORKERS = NUM_CORES * NUM_SUBCORES

SC_TOKENS = 8192
TC_TOKENS = NUM_TOKENS - SC_TOKENS
SLAB = SC_TOKENS // NUM_WORKERS
GROUPS = SLAB // LANES
TOKEN_BLOCK = 2048
WORKERS_PER_BLOCK = TOKEN_BLOCK // SLAB
SC_STEPS = SC_TOKENS // TOKEN_BLOCK
TC_STEPS = TC_TOKENS // TOKEN_BLOCK


def _dot(w, x):
    return lax.dot_general(
        w, x,
        dimension_numbers=(((1,), (1,)), ((), ())),
        preferred_element_type=jnp.float32,
        precision=lax.Precision.DEFAULT,
    )


def _matmul_body(w_ref, x_ref, out_ref):
    out_ref[...] = _dot(w_ref[...], x_ref[...])


def _gate_scores_sc(x, w_gate):
    return pl.pallas_call(
        _matmul_body,
        grid=(SC_STEPS,),
        in_specs=[
            pl.BlockSpec((NUM_EXPERTS, MODEL_DIM), lambda i: (0, 0)),
            pl.BlockSpec((TOKEN_BLOCK, MODEL_DIM), lambda i: (i, 0)),
        ],
        out_specs=pl.BlockSpec((NUM_EXPERTS, TOKEN_BLOCK), lambda i: (0, i)),
        out_shape=jax.ShapeDtypeStruct(
            (NUM_EXPERTS, SC_TOKENS), jnp.float32),
    )(w_gate, x)


def _tc_router_body(w_ref, x_ref, g1_ref, g2_ref, i1_ref, i2_ref):
    res = _dot(w_ref[...], x_ref[...])
    eiota = lax.broadcasted_iota(jnp.int32, res.shape, 0)
    m1 = jnp.max(res, axis=0)
    i1 = jnp.min(jnp.where(res == m1[None, :], eiota, NUM_EXPERTS), axis=0)
    masked = jnp.where(eiota == i1[None, :], -jnp.inf, res)
    m2 = jnp.max(masked, axis=0)
    i2 = jnp.min(jnp.where(masked == m2[None, :], eiota, NUM_EXPERTS), axis=0)
    e2 = jnp.exp(m2 - m1)
    den = 1.0 + e2
    g1_ref[0, 0] = 1.0 / den
    g2_ref[0, 0] = e2 / den
    i1_ref[0, 0] = i1
    i2_ref[0, 0] = i2


def _tc_router(x, w_gate):
    base = SC_TOKENS // TOKEN_BLOCK
    out_spec = pl.BlockSpec((1, 1, TOKEN_BLOCK), lambda i: (i, 0, 0))
    return pl.pallas_call(
        _tc_router_body,
        grid=(TC_STEPS,),
        in_specs=[
            pl.BlockSpec((NUM_EXPERTS, MODEL_DIM), lambda i: (0, 0)),
            pl.BlockSpec((TOKEN_BLOCK, MODEL_DIM), lambda i: (base + i, 0)),
        ],
        out_specs=[out_spec, out_spec, out_spec, out_spec],
        out_shape=(
            jax.ShapeDtypeStruct((TC_STEPS, 1, TOKEN_BLOCK), jnp.float32),
            jax.ShapeDtypeStruct((TC_STEPS, 1, TOKEN_BLOCK), jnp.float32),
            jax.ShapeDtypeStruct((TC_STEPS, 1, TOKEN_BLOCK), jnp.int32),
            jax.ShapeDtypeStruct((TC_STEPS, 1, TOKEN_BLOCK), jnp.int32),
        ),
    )(w_gate, x)


def _sc_topk_body(scores_hbm, g1_hbm, g2_hbm, i1_hbm, i2_hbm,
                  sbuf, g1v, g2v, i1v, i2v):
    cid = lax.axis_index("c")
    sid = lax.axis_index("s")
    wid = sid * NUM_CORES + cid
    pltpu.sync_copy(scores_hbm.at[:, pl.ds(wid * SLAB, SLAB)], sbuf)

    def group(t, carry):
        base = t * LANES
        m1 = jnp.full((LANES,), -jnp.inf, jnp.float32)
        m2 = jnp.full((LANES,), -jnp.inf, jnp.float32)
        i1 = jnp.zeros((LANES,), jnp.int32)
        i2 = jnp.zeros((LANES,), jnp.int32)
        for e in range(NUM_EXPERTS):
            v = sbuf[e, pl.ds(base, LANES)]
            ev = jnp.full((LANES,), e, jnp.int32)
            gt1 = v > m1
            gt2 = v > m2
            i2 = jnp.where(gt1, i1, jnp.where(gt2, ev, i2))
            m2 = jnp.where(gt1, m1, jnp.where(gt2, v, m2))
            i1 = jnp.where(gt1, ev, i1)
            m1 = jnp.where(gt1, v, m1)
        e2 = jnp.exp(m2 - m1)
        den = 1.0 + e2
        sl = pl.ds(base, LANES)
        g1v[sl] = 1.0 / den
        g2v[sl] = e2 / den
        i1v[sl] = i1
        i2v[sl] = i2
        return carry

    lax.fori_loop(0, GROUPS, group, 0)

    rows = pl.ds(wid * SLAB, SLAB)
    pltpu.sync_copy(g1v, g1_hbm.at[rows])
    pltpu.sync_copy(g2v, g2_hbm.at[rows])
    pltpu.sync_copy(i1v, i1_hbm.at[rows])
    pltpu.sync_copy(i2v, i2_hbm.at[rows])


@functools.lru_cache(maxsize=1)
def _sc_topk():
    return pl.kernel(
        _sc_topk_body,
        out_type=(
            jax.ShapeDtypeStruct((SC_TOKENS,), jnp.float32),
            jax.ShapeDtypeStruct((SC_TOKENS,), jnp.float32),
            jax.ShapeDtypeStruct((SC_TOKENS,), jnp.int32),
            jax.ShapeDtypeStruct((SC_TOKENS,), jnp.int32),
        ),
        mesh=plsc.VectorSubcoreMesh(
            core_axis_name="c", subcore_axis_name="s",
            num_cores=NUM_CORES, num_subcores=NUM_SUBCORES),
        scratch_types=(
            pltpu.VMEM((NUM_EXPERTS, SLAB), jnp.float32),
            pltpu.VMEM((SLAB,), jnp.float32),
            pltpu.VMEM((SLAB,), jnp.float32),
            pltpu.VMEM((SLAB,), jnp.int32),
            pltpu.VMEM((SLAB,), jnp.int32),
        ),
    )


def kernel(x, W_gate):
    scores_sc = _gate_scores_sc(x, W_gate)
    sc_g1, sc_g2, sc_i1, sc_i2 = _sc_topk()(scores_sc)
    tc_g1, tc_g2, tc_i1, tc_i2 = (a.reshape(TC_TOKENS) for a in _tc_router(x, W_gate))
    g1 = jnp.concatenate([sc_g1, tc_g1])
    g2 = jnp.concatenate([sc_g2, tc_g2])
    i1 = jnp.concatenate([sc_i1, tc_i1])
    i2 = jnp.concatenate([sc_i2, tc_i2])
    top_k_gates = jnp.stack([g1, g2], axis=-1)
    top_k_indices = jnp.stack([i1, i2], axis=-1)
    return top_k_gates, top_k_indices


# speedup vs baseline: 1.0449x; 1.0449x over previous
import functools

import jax
import jax.numpy as jnp
from jax import lax
from jax.experimental import pallas as pl
from jax.experimental.pallas import tpu as pltpu
from jax.experimental.pallas import tpu_sc as plsc

NUM_TOKENS = 16384
MODEL_DIM = 2048
NUM_EXPERTS = 64
LANES = 16
NUM_CORES = 2
NUM_SUBCORES = 16
NUM_WORKERS = NUM_CORES * NUM_SUBCORES

SC_TOKENS = 8192
TC_TOKENS = NUM_TOKENS - SC_TOKENS
SLAB = SC_TOKENS // NUM_WORKERS
GROUPS = SLAB // LANES
TOKEN_BLOCK = 1024
WORKERS_PER_BLOCK = TOKEN_BLOCK // SLAB
SC_STEPS = SC_TOKENS // TOKEN_BLOCK
TC_STEPS = TC_TOKENS // TOKEN_BLOCK


def _dot(w, x):
    return lax.dot_general(
        w, x,
        dimension_numbers=(((1,), (1,)), ((), ())),
        preferred_element_type=jnp.float32,
        precision=lax.Precision.DEFAULT,
    )


def _matmul_body(w_ref, x_ref, out_ref):
    out_ref[...] = _dot(w_ref[...], x_ref[...])


def _gate_scores_sc(x, w_gate):
    return pl.pallas_call(
        _matmul_body,
        grid=(SC_STEPS,),
        in_specs=[
            pl.BlockSpec((NUM_EXPERTS, MODEL_DIM), lambda i: (0, 0)),
            pl.BlockSpec((TOKEN_BLOCK, MODEL_DIM), lambda i: (i, 0)),
        ],
        out_specs=pl.BlockSpec((NUM_EXPERTS, TOKEN_BLOCK), lambda i: (0, i)),
        out_shape=jax.ShapeDtypeStruct(
            (NUM_EXPERTS, SC_TOKENS), jnp.float32),
    )(w_gate, x)


def _tc_router_body(w_ref, x_ref, g1_ref, g2_ref, i1_ref, i2_ref):
    res = _dot(w_ref[...], x_ref[...])
    eiota = lax.broadcasted_iota(jnp.int32, res.shape, 0)
    m1 = jnp.max(res, axis=0)
    i1 = jnp.min(jnp.where(res == m1[None, :], eiota, NUM_EXPERTS), axis=0)
    masked = jnp.where(eiota == i1[None, :], -jnp.inf, res)
    m2 = jnp.max(masked, axis=0)
    i2 = jnp.min(jnp.where(masked == m2[None, :], eiota, NUM_EXPERTS), axis=0)
    e2 = jnp.exp(m2 - m1)
    den = 1.0 + e2
    g1_ref[0, 0] = 1.0 / den
    g2_ref[0, 0] = e2 / den
    i1_ref[0, 0] = i1
    i2_ref[0, 0] = i2


def _tc_router(x, w_gate):
    base = SC_TOKENS // TOKEN_BLOCK
    out_spec = pl.BlockSpec((1, 1, TOKEN_BLOCK), lambda i: (i, 0, 0))
    return pl.pallas_call(
        _tc_router_body,
        grid=(TC_STEPS,),
        in_specs=[
            pl.BlockSpec((NUM_EXPERTS, MODEL_DIM), lambda i: (0, 0)),
            pl.BlockSpec((TOKEN_BLOCK, MODEL_DIM), lambda i: (base + i, 0)),
        ],
        out_specs=[out_spec, out_spec, out_spec, out_spec],
        out_shape=(
            jax.ShapeDtypeStruct((TC_STEPS, 1, TOKEN_BLOCK), jnp.float32),
            jax.ShapeDtypeStruct((TC_STEPS, 1, TOKEN_BLOCK), jnp.float32),
            jax.ShapeDtypeStruct((TC_STEPS, 1, TOKEN_BLOCK), jnp.int32),
            jax.ShapeDtypeStruct((TC_STEPS, 1, TOKEN_BLOCK), jnp.int32),
        ),
    )(w_gate, x)


def _sc_topk_body(scores_hbm, g1_hbm, g2_hbm, i1_hbm, i2_hbm,
                  sbuf, g1v, g2v, i1v, i2v):
    cid = lax.axis_index("c")
    sid = lax.axis_index("s")
    wid = sid * NUM_CORES + cid
    pltpu.sync_copy(scores_hbm.at[:, pl.ds(wid * SLAB, SLAB)], sbuf)

    def group(t, carry):
        base = t * LANES
        m1 = jnp.full((LANES,), -jnp.inf, jnp.float32)
        m2 = jnp.full((LANES,), -jnp.inf, jnp.float32)
        i1 = jnp.zeros((LANES,), jnp.int32)
        i2 = jnp.zeros((LANES,), jnp.int32)
        for e in range(NUM_EXPERTS):
            v = sbuf[e, pl.ds(base, LANES)]
            ev = jnp.full((LANES,), e, jnp.int32)
            gt1 = v > m1
            gt2 = v > m2
            i2 = jnp.where(gt1, i1, jnp.where(gt2, ev, i2))
            m2 = jnp.where(gt1, m1, jnp.where(gt2, v, m2))
            i1 = jnp.where(gt1, ev, i1)
            m1 = jnp.where(gt1, v, m1)
        e2 = jnp.exp(m2 - m1)
        den = 1.0 + e2
        sl = pl.ds(base, LANES)
        g1v[sl] = 1.0 / den
        g2v[sl] = e2 / den
        i1v[sl] = i1
        i2v[sl] = i2
        return carry

    lax.fori_loop(0, GROUPS, group, 0)

    rows = pl.ds(wid * SLAB, SLAB)
    pltpu.sync_copy(g1v, g1_hbm.at[rows])
    pltpu.sync_copy(g2v, g2_hbm.at[rows])
    pltpu.sync_copy(i1v, i1_hbm.at[rows])
    pltpu.sync_copy(i2v, i2_hbm.at[rows])


@functools.lru_cache(maxsize=1)
def _sc_topk():
    return pl.kernel(
        _sc_topk_body,
        out_type=(
            jax.ShapeDtypeStruct((SC_TOKENS,), jnp.float32),
            jax.ShapeDtypeStruct((SC_TOKENS,), jnp.float32),
            jax.ShapeDtypeStruct((SC_TOKENS,), jnp.int32),
            jax.ShapeDtypeStruct((SC_TOKENS,), jnp.int32),
        ),
        mesh=plsc.VectorSubcoreMesh(
            core_axis_name="c", subcore_axis_name="s",
            num_cores=NUM_CORES, num_subcores=NUM_SUBCORES),
        scratch_types=(
            pltpu.VMEM((NUM_EXPERTS, SLAB), jnp.float32),
            pltpu.VMEM((SLAB,), jnp.float32),
            pltpu.VMEM((SLAB,), jnp.float32),
            pltpu.VMEM((SLAB,), jnp.int32),
            pltpu.VMEM((SLAB,), jnp.int32),
        ),
    )


def kernel(x, W_gate):
    scores_sc = _gate_scores_sc(x, W_gate)
    sc_g1, sc_g2, sc_i1, sc_i2 = _sc_topk()(scores_sc)
    tc_g1, tc_g2, tc_i1, tc_i2 = (a.reshape(TC_TOKENS) for a in _tc_router(x, W_gate))
    g1 = jnp.concatenate([sc_g1, tc_g1])
    g2 = jnp.concatenate([sc_g2, tc_g2])
    i1 = jnp.concatenate([sc_i1, tc_i1])
    i2 = jnp.concatenate([sc_i2, tc_i2])
    top_k_gates = jnp.stack([g1, g2], axis=-1)
    top_k_indices = jnp.stack([i1, i2], axis=-1)
    return top_k_gates, top_k_indices
